# Initial kernel scaffold; baseline (speedup 1.0000x reference)
import functools

import jax
import jax.numpy as jnp
from jax import lax
from jax.experimental import pallas as pl
from jax.experimental.pallas import tpu as pltpu
from jax.experimental.pallas import tpu_sc as plsc

N_ROWS = 100000
F = 128
NSEG = 512
NC = 2
NS = 16
L = 16
NW = NC * NS
SEG_PER_W = NSEG // NW
SLICE = N_ROWS // NS
SLICE_PAD = SLICE + 8 - (SLICE % 8)
NTGT = NSEG + L
BLK = 128
FV = F // L


def _body(x_hbm, batch_hbm, out_hbm,
          bslice, local_ss, all_ss, off, buf, stage, shared_ss):
  cid = lax.axis_index("c")
  sid = lax.axis_index("s")
  w = sid * NC + cid

  raw = sid * SLICE
  a8 = (raw // 8) * 8
  shift = raw - a8
  pltpu.sync_copy(batch_hbm.at[pl.ds(a8, SLICE_PAD)], bslice)

  for g in range(NTGT // L):
    tgt = jnp.int32(g * L) + lax.iota(jnp.int32, (L,))
    lo = jnp.zeros((L,), jnp.int32)
    hi = jnp.full((L,), SLICE, jnp.int32)
    for _ in range(13):
      active = lo < hi
      mid = lax.shift_right_logical(lo + hi, 1)
      v = plsc.load_gather(bslice, [mid + shift])
      down = jnp.logical_and(active, v < tgt)
      lo = jnp.where(down, mid + 1, lo)
      hi = jnp.where(jnp.logical_and(active, jnp.logical_not(down)), mid, hi)
    local_ss[pl.ds(g * L, L)] = lo

  pltpu.sync_copy(local_ss, shared_ss.at[sid])
  plsc.subcore_barrier()
  pltpu.sync_copy(shared_ss, all_ss)
  for g in range(NTGT // L):
    acc = jnp.zeros((L,), jnp.int32)
    for k in range(NS):
      acc = acc + all_ss[k, pl.ds(g * L, L)]
    off[pl.ds(g * L, L)] = acc

  zero = jnp.zeros((L,), jnp.float32)
  ninf = jnp.full((L,), -jnp.inf, jnp.float32)

  def seg_step(i, _):
    s = w * SEG_PER_W + i
    a = off[s]
    b = off[s + 1]
    c = b - a
    nblk = lax.div(c + (BLK - 1), BLK)

    def blk_loop(k, carry):
      sums, maxs = carry
      start0 = a + k * BLK
      start = jnp.minimum(start0, N_ROWS - BLK)
      pltpu.sync_copy(x_hbm.at[pl.ds(start, BLK)], buf)
      rlo = start0 - start
      rhi = jnp.minimum(start0 + BLK, b) - start

      def row_loop(r, rc):
        rs, rm = rc
        row = [buf[r, pl.ds(j * L, L)] for j in range(FV)]
        rs = tuple(rs[j] + row[j] for j in range(FV))
        rm = tuple(jnp.maximum(rm[j], row[j]) for j in range(FV))
        return rs, rm

      return lax.fori_loop(rlo, rhi, row_loop, (sums, maxs))

    sums, maxs = lax.fori_loop(
        0, nblk, blk_loop,
        (tuple(zero for _ in range(FV)), tuple(ninf for _ in range(FV))))

    rcp = 1.0 / jnp.maximum(c, 1).astype(jnp.float32)
    nonempty = c > 0
    for j in range(FV):
      stage[i, pl.ds(j * L, L)] = sums[j] * rcp
      stage[i, pl.ds(F + j * L, L)] = jnp.where(nonempty, maxs[j], 0.0)
    return 0

  lax.fori_loop(0, SEG_PER_W, seg_step, 0)
  pltpu.sync_copy(stage, out_hbm.at[pl.ds(w * SEG_PER_W, SEG_PER_W)])


@functools.partial(jax.jit, static_argnames=())
def _pooled(x, batch):
  mesh = plsc.VectorSubcoreMesh(core_axis_name="c", subcore_axis_name="s",
                                num_cores=NC, num_subcores=NS)
  fn = pl.kernel(
      _body,
      out_type=jax.ShapeDtypeStruct((NSEG, 2 * F), jnp.float32),
      mesh=mesh,
      scratch_types=[
          pltpu.VMEM((SLICE_PAD,), jnp.int32),
          pltpu.VMEM((NTGT,), jnp.int32),
          pltpu.VMEM((NS, NTGT), jnp.int32),
          pltpu.VMEM((NTGT,), jnp.int32),
          pltpu.VMEM((BLK, F), jnp.float32),
          pltpu.VMEM((SEG_PER_W, 2 * F), jnp.float32),
          pltpu.MemorySpace.VMEM_SHARED((NS, NTGT), jnp.int32),
      ],
  )
  return fn(x, batch)


def kernel(x, batch):
  return _pooled(x, batch.astype(jnp.int32))


# speedup vs baseline: 8.1543x; 8.1543x over previous
import functools

import jax
import jax.numpy as jnp
from jax import lax
from jax.experimental import pallas as pl
from jax.experimental.pallas import tpu as pltpu
from jax.experimental.pallas import tpu_sc as plsc

N_ROWS = 100000
F = 128
NSEG = 512
NC = 2
NS = 16
L = 16
NW = NC * NS
SEG_PER_W = NSEG // NW
NSLC = 8
SLICE = N_ROWS // NSLC
SLICE_PAD = 12544
BATCH_LEN = N_ROWS + 64
NITER = 14
BLK = 128
FV = F // L


def _gather16(ref, idx):
  return plsc.load_gather(ref, [idx])


def _body(x_hbm, batch_hbm, out_hbm, bslice, off, buf, stage):
  cid = lax.axis_index("c")
  sid = lax.axis_index("s")
  w = sid * NC + cid
  s0 = w * SEG_PER_W
  iota = lax.iota(jnp.int32, L)

  tgt_a = s0 + iota
  tgt_b = tgt_a + L
  off_a = jnp.zeros((L,), jnp.int32)
  off_b = jnp.zeros((L,), jnp.int32)

  for k in range(NSLC):
    raw = k * SLICE
    a8 = (raw // 8) * 8
    shift = raw - a8
    pltpu.sync_copy(batch_hbm.at[pl.ds(a8, SLICE_PAD)], bslice)

    def lower_bound(tgt):
      lo = jnp.zeros((L,), jnp.int32)
      hi = jnp.full((L,), SLICE, jnp.int32)
      for _ in range(NITER):
        active = lo < hi
        mid = lax.shift_right_logical(lo + hi, 1)
        v = _gather16(bslice, mid + shift)
        down = jnp.logical_and(active, v < tgt)
        lo = jnp.where(down, mid + 1, lo)
        hi = jnp.where(jnp.logical_and(active, jnp.logical_not(down)), mid, hi)
      return lo

    off_a = off_a + lower_bound(tgt_a)
    off_b = off_b + lower_bound(tgt_b)

  off[pl.ds(0, L)] = off_a
  off[pl.ds(L, L)] = off_b

  zero = jnp.zeros((L,), jnp.float32)
  ninf = jnp.full((L,), -jnp.inf, jnp.float32)

  def seg_step(i, _):
    ovec = off[pl.ds(i, L)]
    a = ovec[0]
    b = ovec[1]
    c = b - a
    astart = (a // 8) * 8
    nblk = lax.div(b - astart + (BLK - 1), BLK)

    def blk_loop(k, carry):
      sums, maxs = carry
      start0 = astart + k * BLK
      start = jnp.minimum(start0, N_ROWS - BLK)
      pltpu.sync_copy(x_hbm.at[pl.ds(start, BLK)], buf)
      rlo = jnp.maximum(a, start0) - start
      rhi = jnp.minimum(start0 + BLK, b) - start

      def row_loop(r, rc):
        rs, rm = rc
        row = [buf[r, pl.ds(j * L, L)] for j in range(FV)]
        rs = tuple(rs[j] + row[j] for j in range(FV))
        rm = tuple(jnp.maximum(rm[j], row[j]) for j in range(FV))
        return rs, rm

      return lax.fori_loop(rlo, rhi, row_loop, (sums, maxs))

    sums, maxs = lax.fori_loop(
        0, nblk, blk_loop,
        (tuple(zero for _ in range(FV)), tuple(ninf for _ in range(FV))))

    cvec = jnp.full((L,), 1.0, jnp.float32) * jnp.maximum(c, 1).astype(jnp.float32)
    rcp = jnp.full((L,), 1.0, jnp.float32) / cvec
    nonempty = c > 0
    for j in range(FV):
      stage[i, pl.ds(j * L, L)] = sums[j] * rcp
      stage[i, pl.ds(F + j * L, L)] = jnp.where(nonempty, maxs[j], 0.0)
    return 0

  lax.fori_loop(0, SEG_PER_W, seg_step, 0)
  pltpu.sync_copy(stage, out_hbm.at[pl.ds(s0, SEG_PER_W)])


@jax.jit
def _pooled(x, batch):
  mesh = plsc.VectorSubcoreMesh(core_axis_name="c", subcore_axis_name="s",
                                num_cores=NC, num_subcores=NS)
  fn = pl.kernel(
      _body,
      out_type=jax.ShapeDtypeStruct((NSEG, 2 * F), jnp.float32),
      mesh=mesh,
      scratch_types=[
          pltpu.VMEM((SLICE_PAD,), jnp.int32),
          pltpu.VMEM((2 * L,), jnp.int32),
          pltpu.VMEM((BLK, F), jnp.float32),
          pltpu.VMEM((SEG_PER_W, 2 * F), jnp.float32),
      ],
      compiler_params=pltpu.CompilerParams(needs_layout_passes=False),
  )
  return fn(x, batch)


def kernel(x, batch):
  bpad = jnp.pad(batch.astype(jnp.int32), (0, BATCH_LEN - N_ROWS))
  return _pooled(x, bpad)


# speedup vs baseline: 10.7685x; 1.3206x over previous
import functools

import jax
import jax.numpy as jnp
from jax import lax
from jax.experimental import pallas as pl
from jax.experimental.pallas import tpu as pltpu
from jax.experimental.pallas import tpu_sc as plsc

N_ROWS = 100000
F = 128
NSEG = 512
NC = 2
NS = 16
L = 16
NW = NC * NS
SEG_PER_W = NSEG // NW
NSLC = 8
SLICE = N_ROWS // NSLC
SLICE_PAD = 12544
BATCH_LEN = N_ROWS + 64
NITER = 14
BLK = 128
FV = F // L


def _gather16(ref, idx):
  return plsc.load_gather(ref, [idx])


def _body(x_hbm, batch_hbm, out_hbm, bslice, off, buf0, buf1, stage,
          sem0, sem1):
  cid = lax.axis_index("c")
  sid = lax.axis_index("s")
  w = sid * NC + cid
  s0 = w * SEG_PER_W
  iota = lax.iota(jnp.int32, L)

  tgt_a = s0 + iota
  tgt_b = tgt_a + L
  off_a = jnp.zeros((L,), jnp.int32)
  off_b = jnp.zeros((L,), jnp.int32)

  for k in range(NSLC):
    raw = k * SLICE
    a8 = (raw // 8) * 8
    shift = raw - a8
    pltpu.sync_copy(batch_hbm.at[pl.ds(a8, SLICE_PAD)], bslice)

    def lower_bound(tgt):
      lo = jnp.zeros((L,), jnp.int32)
      hi = jnp.full((L,), SLICE, jnp.int32)
      for _ in range(NITER):
        active = lo < hi
        mid = lax.shift_right_logical(lo + hi, 1)
        v = _gather16(bslice, mid + shift)
        down = jnp.logical_and(active, v < tgt)
        lo = jnp.where(down, mid + 1, lo)
        hi = jnp.where(jnp.logical_and(active, jnp.logical_not(down)), mid, hi)
      return lo

    off_a = off_a + lower_bound(tgt_a)
    off_b = off_b + lower_bound(tgt_b)

  off[pl.ds(0, L)] = off_a
  off[pl.ds(L, L)] = off_b

  zero = jnp.zeros((L,), jnp.float32)
  ninf = jnp.full((L,), -jnp.inf, jnp.float32)
  a_all = off_a[0]
  b_all = off_b[0]
  astart = (a_all // 8) * 8
  nblk = jnp.maximum(lax.div(b_all - astart + (BLK - 1), BLK), 1)

  bufs = (buf0, buf1)
  sems = (sem0, sem1)

  def fire(k, b):
    start = jnp.minimum(astart + k * BLK, N_ROWS - BLK)
    pltpu.async_copy(x_hbm.at[pl.ds(start, BLK)], bufs[b], sems[b])

  @pl.when(nblk > 0)
  def _():
    fire(jnp.int32(0), 0)

  def accum_rows(buf, rlo, rhi, sums, maxs):
    def row_loop(r, rc):
      rs, rm = rc
      row = [buf[r, pl.ds(j * L, L)] for j in range(FV)]
      return (tuple(rs[j] + row[j] for j in range(FV)),
              tuple(jnp.maximum(rm[j], row[j]) for j in range(FV)))
    return lax.fori_loop(rlo, rhi, row_loop, (sums, maxs))

  def process_block(k, b, carry):
    i, g, sums, maxs = carry
    start0 = astart + k * BLK
    start = jnp.minimum(start0, N_ROWS - BLK)
    buf = bufs[b]
    pltpu.make_async_copy(x_hbm.at[pl.ds(start, BLK)], buf, sems[b]).wait()

    @pl.when(k + 1 < nblk)
    def _():
      fire(k + 1, 1 - b)

    ghi = jnp.minimum(start0 + BLK, b_all)

    def seg_cond(st):
      i, g, sums, maxs = st
      b_i = off[pl.ds(i + 1, L)][0]
      return jnp.logical_and(i < SEG_PER_W, b_i <= ghi)

    def seg_fin(st):
      i, g, sums, maxs = st
      b_i = off[pl.ds(i + 1, L)][0]
      sums, maxs = accum_rows(buf, g - start, b_i - start, sums, maxs)
      c = b_i - off[pl.ds(i, L)][0]
      cvec = jnp.full((L,), 1.0, jnp.float32) * jnp.maximum(c, 1).astype(jnp.float32)
      rcp = jnp.full((L,), 1.0, jnp.float32) / cvec
      nonempty = c > 0
      for j in range(FV):
        stage[i, pl.ds(j * L, L)] = sums[j] * rcp
        stage[i, pl.ds(F + j * L, L)] = jnp.where(nonempty, maxs[j], 0.0)
      return (i + 1, b_i,
              tuple(zero for _ in range(FV)), tuple(ninf for _ in range(FV)))

    i, g, sums, maxs = lax.while_loop(seg_cond, seg_fin, (i, g, sums, maxs))
    sums, maxs = accum_rows(buf, jnp.maximum(g, start0) - start, ghi - start,
                            sums, maxs)
    return (i, jnp.maximum(g, ghi), sums, maxs)

  def blk_pair(k2, carry):
    k = k2 * 2
    carry = lax.cond(k < nblk,
                     lambda c: process_block(k, 0, c), lambda c: c, carry)
    carry = lax.cond(k + 1 < nblk,
                     lambda c: process_block(k + 1, 1, c), lambda c: c, carry)
    return carry

  carry0 = (jnp.int32(0), a_all,
            tuple(zero for _ in range(FV)), tuple(ninf for _ in range(FV)))
  npair = lax.div(nblk + 1, 2)
  lax.fori_loop(0, npair, blk_pair, carry0)

  pltpu.sync_copy(stage, out_hbm.at[pl.ds(s0, SEG_PER_W)])


@jax.jit
def _pooled(x, batch):
  mesh = plsc.VectorSubcoreMesh(core_axis_name="c", subcore_axis_name="s",
                                num_cores=NC, num_subcores=NS)
  fn = pl.kernel(
      _body,
      out_type=jax.ShapeDtypeStruct((NSEG, 2 * F), jnp.float32),
      mesh=mesh,
      scratch_types=[
          pltpu.VMEM((SLICE_PAD,), jnp.int32),
          pltpu.VMEM((2 * L,), jnp.int32),
          pltpu.VMEM((BLK, F), jnp.float32),
          pltpu.VMEM((BLK, F), jnp.float32),
          pltpu.VMEM((SEG_PER_W, 2 * F), jnp.float32),
          pltpu.SemaphoreType.DMA,
          pltpu.SemaphoreType.DMA,
      ],
      compiler_params=pltpu.CompilerParams(needs_layout_passes=False),
  )
  return fn(x, batch)


def kernel(x, batch):
  bpad = jnp.pad(batch.astype(jnp.int32), (0, BATCH_LEN - N_ROWS))
  return _pooled(x, bpad)


# speedup vs baseline: 11.9225x; 1.1072x over previous
import functools

import jax
import jax.numpy as jnp
from jax import lax
from jax.experimental import pallas as pl
from jax.experimental.pallas import tpu as pltpu
from jax.experimental.pallas import tpu_sc as plsc

N_ROWS = 100000
F = 128
NSEG = 512
NC = 2
NS = 16
L = 16
NW = NC * NS
SEG_PER_W = NSEG // NW
NSLC = 8
SLICE = N_ROWS // NSLC
SLICE_PAD = 12544
BATCH_LEN = N_ROWS + 64
NITER = 14
BLK = 256
FV = F // L


def _gather16(ref, idx):
  return plsc.load_gather(ref, [idx])


def _body(x_hbm, batch_hbm, out_hbm, bslice, off, buf0, buf1, stage,
          sem0, sem1):
  cid = lax.axis_index("c")
  sid = lax.axis_index("s")
  w = sid * NC + cid
  s0 = w * SEG_PER_W
  iota = lax.iota(jnp.int32, L)

  tgt_a = s0 + iota
  tgt_b = tgt_a + L
  off_a = jnp.zeros((L,), jnp.int32)
  off_b = jnp.zeros((L,), jnp.int32)

  for k in range(NSLC):
    raw = k * SLICE
    a8 = (raw // 8) * 8
    shift = raw - a8
    pltpu.sync_copy(batch_hbm.at[pl.ds(a8, SLICE_PAD)], bslice)

    def lower_bound(tgt):
      lo = jnp.zeros((L,), jnp.int32)
      hi = jnp.full((L,), SLICE, jnp.int32)
      for _ in range(NITER):
        active = lo < hi
        mid = lax.shift_right_logical(lo + hi, 1)
        v = _gather16(bslice, mid + shift)
        down = jnp.logical_and(active, v < tgt)
        lo = jnp.where(down, mid + 1, lo)
        hi = jnp.where(jnp.logical_and(active, jnp.logical_not(down)), mid, hi)
      return lo

    off_a = off_a + lower_bound(tgt_a)
    off_b = off_b + lower_bound(tgt_b)

  off[pl.ds(0, L)] = off_a
  off[pl.ds(L, L)] = off_b

  zero = jnp.zeros((L,), jnp.float32)
  ninf = jnp.full((L,), -jnp.inf, jnp.float32)
  a_all = off_a[0]
  b_all = off_b[0]
  astart = (a_all // 8) * 8
  nblk = jnp.maximum(lax.div(b_all - astart + (BLK - 1), BLK), 1)

  bufs = (buf0, buf1)
  sems = (sem0, sem1)

  def fire(k, b):
    start = jnp.minimum(astart + k * BLK, N_ROWS - BLK)
    pltpu.async_copy(x_hbm.at[pl.ds(start, BLK)], bufs[b], sems[b])

  @pl.when(nblk > 0)
  def _():
    fire(jnp.int32(0), 0)

  def accum_rows(buf, rlo, rhi, sums, maxs):
    n = jnp.maximum(rhi - rlo, 0)

    def pair_loop(p, rc):
      rs, rm = rc
      r = rlo + p * 2
      row0 = [buf[r, pl.ds(j * L, L)] for j in range(FV)]
      row1 = [buf[r + 1, pl.ds(j * L, L)] for j in range(FV)]
      return (tuple(rs[j] + (row0[j] + row1[j]) for j in range(FV)),
              tuple(jnp.maximum(rm[j], jnp.maximum(row0[j], row1[j]))
                    for j in range(FV)))

    sums, maxs = lax.fori_loop(0, lax.shift_right_logical(n, 1), pair_loop,
                               (sums, maxs))

    def odd_tail(rc):
      rs, rm = rc
      row = [buf[rhi - 1, pl.ds(j * L, L)] for j in range(FV)]
      return (tuple(rs[j] + row[j] for j in range(FV)),
              tuple(jnp.maximum(rm[j], row[j]) for j in range(FV)))

    return lax.cond((n & 1) > 0, odd_tail, lambda rc: rc, (sums, maxs))

  def process_block(k, b, carry):
    i, g, sums, maxs = carry
    start0 = astart + k * BLK
    start = jnp.minimum(start0, N_ROWS - BLK)
    buf = bufs[b]
    pltpu.make_async_copy(x_hbm.at[pl.ds(start, BLK)], buf, sems[b]).wait()

    @pl.when(k + 1 < nblk)
    def _():
      fire(k + 1, 1 - b)

    ghi = jnp.minimum(start0 + BLK, b_all)

    def seg_cond(st):
      i, g, sums, maxs = st
      b_i = off[pl.ds(i + 1, L)][0]
      return jnp.logical_and(i < SEG_PER_W, b_i <= ghi)

    def seg_fin(st):
      i, g, sums, maxs = st
      b_i = off[pl.ds(i + 1, L)][0]
      sums, maxs = accum_rows(buf, g - start, b_i - start, sums, maxs)
      c = b_i - off[pl.ds(i, L)][0]
      cvec = jnp.full((L,), 1.0, jnp.float32) * jnp.maximum(c, 1).astype(jnp.float32)
      rcp = jnp.full((L,), 1.0, jnp.float32) / cvec
      nonempty = c > 0
      for j in range(FV):
        stage[i, pl.ds(j * L, L)] = sums[j] * rcp
        stage[i, pl.ds(F + j * L, L)] = jnp.where(nonempty, maxs[j], 0.0)
      return (i + 1, b_i,
              tuple(zero for _ in range(FV)), tuple(ninf for _ in range(FV)))

    i, g, sums, maxs = lax.while_loop(seg_cond, seg_fin, (i, g, sums, maxs))
    sums, maxs = accum_rows(buf, jnp.maximum(g, start0) - start, ghi - start,
                            sums, maxs)
    return (i, jnp.maximum(g, ghi), sums, maxs)

  def blk_pair(k2, carry):
    k = k2 * 2
    carry = lax.cond(k < nblk,
                     lambda c: process_block(k, 0, c), lambda c: c, carry)
    carry = lax.cond(k + 1 < nblk,
                     lambda c: process_block(k + 1, 1, c), lambda c: c, carry)
    return carry

  carry0 = (jnp.int32(0), a_all,
            tuple(zero for _ in range(FV)), tuple(ninf for _ in range(FV)))
  npair = lax.div(nblk + 1, 2)
  lax.fori_loop(0, npair, blk_pair, carry0)

  pltpu.sync_copy(stage, out_hbm.at[pl.ds(s0, SEG_PER_W)])


@jax.jit
def _pooled(x, batch):
  mesh = plsc.VectorSubcoreMesh(core_axis_name="c", subcore_axis_name="s",
                                num_cores=NC, num_subcores=NS)
  fn = pl.kernel(
      _body,
      out_type=jax.ShapeDtypeStruct((NSEG, 2 * F), jnp.float32),
      mesh=mesh,
      scratch_types=[
          pltpu.VMEM((SLICE_PAD,), jnp.int32),
          pltpu.VMEM((2 * L,), jnp.int32),
          pltpu.VMEM((BLK, F), jnp.float32),
          pltpu.VMEM((BLK, F), jnp.float32),
          pltpu.VMEM((SEG_PER_W, 2 * F), jnp.float32),
          pltpu.SemaphoreType.DMA,
          pltpu.SemaphoreType.DMA,
      ],
      compiler_params=pltpu.CompilerParams(needs_layout_passes=False),
  )
  return fn(x, batch)


def kernel(x, batch):
  bpad = jnp.pad(batch.astype(jnp.int32), (0, BATCH_LEN - N_ROWS))
  return _pooled(x, bpad)


# speedup vs baseline: 12.0871x; 1.0138x over previous
import functools

import jax
import jax.numpy as jnp
from jax import lax
from jax.experimental import pallas as pl
from jax.experimental.pallas import tpu as pltpu
from jax.experimental.pallas import tpu_sc as plsc

N_ROWS = 100000
F = 128
NSEG = 512
NC = 2
NS = 16
L = 16
NW = NC * NS
SEG_PER_W = NSEG // NW
NSLC = 8
SLICE = N_ROWS // NSLC
SLICE_PAD = 12544
BATCH_LEN = N_ROWS + 64
NITER = 14
BLK = 256
FV = F // L


def _gather16(ref, idx):
  return plsc.load_gather(ref, [idx])


def _body(x_hbm, batch_hbm, out_hbm, bslice0, bslice1, off, buf0, buf1, stage,
          sem0, sem1):
  cid = lax.axis_index("c")
  sid = lax.axis_index("s")
  w = sid * NC + cid
  s0 = w * SEG_PER_W
  iota = lax.iota(jnp.int32, L)

  tgt_a = s0 + iota
  tgt_b = tgt_a + L
  off_a = jnp.zeros((L,), jnp.int32)
  off_b = jnp.zeros((L,), jnp.int32)

  bslices = (bslice0, bslice1)
  psems = (sem0, sem1)

  def slice_window(k):
    raw = k * SLICE
    a8 = (raw // 8) * 8
    return a8, raw - a8

  a80, _ = slice_window(0)
  pltpu.async_copy(batch_hbm.at[pl.ds(a80, SLICE_PAD)], bslices[0], psems[0])
  for k in range(NSLC):
    p = k & 1
    a8, shift = slice_window(k)
    pltpu.make_async_copy(batch_hbm.at[pl.ds(a8, SLICE_PAD)],
                          bslices[p], psems[p]).wait()
    if k + 1 < NSLC:
      a8n, _ = slice_window(k + 1)
      pltpu.async_copy(batch_hbm.at[pl.ds(a8n, SLICE_PAD)],
                       bslices[1 - p], psems[1 - p])
    bslice = bslices[p]

    def lower_bound(tgt):
      lo = jnp.zeros((L,), jnp.int32)
      hi = jnp.full((L,), SLICE, jnp.int32)
      for _ in range(NITER):
        active = lo < hi
        mid = lax.shift_right_logical(lo + hi, 1)
        v = _gather16(bslice, mid + shift)
        down = jnp.logical_and(active, v < tgt)
        lo = jnp.where(down, mid + 1, lo)
        hi = jnp.where(jnp.logical_and(active, jnp.logical_not(down)), mid, hi)
      return lo

    off_a = off_a + lower_bound(tgt_a)
    off_b = off_b + lower_bound(tgt_b)

  off[pl.ds(0, L)] = off_a
  off[pl.ds(L, L)] = off_b

  zero = jnp.zeros((L,), jnp.float32)
  ninf = jnp.full((L,), -jnp.inf, jnp.float32)
  a_all = off_a[0]
  b_all = off_b[0]
  astart = (a_all // 8) * 8
  nblk = jnp.maximum(lax.div(b_all - astart + (BLK - 1), BLK), 1)

  bufs = (buf0, buf1)
  sems = (sem0, sem1)

  def fire(k, b):
    start = jnp.minimum(astart + k * BLK, N_ROWS - BLK)
    pltpu.async_copy(x_hbm.at[pl.ds(start, BLK)], bufs[b], sems[b])

  @pl.when(nblk > 0)
  def _():
    fire(jnp.int32(0), 0)

  def accum_rows(buf, rlo, rhi, sums, maxs):
    n = jnp.maximum(rhi - rlo, 0)

    def pair_loop(p, rc):
      rs, rm = rc
      r = rlo + p * 2
      row0 = [buf[r, pl.ds(j * L, L)] for j in range(FV)]
      row1 = [buf[r + 1, pl.ds(j * L, L)] for j in range(FV)]
      return (tuple(rs[j] + (row0[j] + row1[j]) for j in range(FV)),
              tuple(jnp.maximum(rm[j], jnp.maximum(row0[j], row1[j]))
                    for j in range(FV)))

    sums, maxs = lax.fori_loop(0, lax.shift_right_logical(n, 1), pair_loop,
                               (sums, maxs))

    def odd_tail(rc):
      rs, rm = rc
      row = [buf[rhi - 1, pl.ds(j * L, L)] for j in range(FV)]
      return (tuple(rs[j] + row[j] for j in range(FV)),
              tuple(jnp.maximum(rm[j], row[j]) for j in range(FV)))

    return lax.cond((n & 1) > 0, odd_tail, lambda rc: rc, (sums, maxs))

  def process_block(k, b, carry):
    i, g, sums, maxs = carry
    start0 = astart + k * BLK
    start = jnp.minimum(start0, N_ROWS - BLK)
    buf = bufs[b]
    pltpu.make_async_copy(x_hbm.at[pl.ds(start, BLK)], buf, sems[b]).wait()

    @pl.when(k + 1 < nblk)
    def _():
      fire(k + 1, 1 - b)

    ghi = jnp.minimum(start0 + BLK, b_all)

    def seg_cond(st):
      i, g, sums, maxs = st
      b_i = off[pl.ds(i + 1, L)][0]
      return jnp.logical_and(i < SEG_PER_W, b_i <= ghi)

    def seg_fin(st):
      i, g, sums, maxs = st
      b_i = off[pl.ds(i + 1, L)][0]
      sums, maxs = accum_rows(buf, g - start, b_i - start, sums, maxs)
      c = b_i - off[pl.ds(i, L)][0]
      cvec = jnp.full((L,), 1.0, jnp.float32) * jnp.maximum(c, 1).astype(jnp.float32)
      rcp = jnp.full((L,), 1.0, jnp.float32) / cvec
      nonempty = c > 0
      for j in range(FV):
        stage[i, pl.ds(j * L, L)] = sums[j] * rcp
        stage[i, pl.ds(F + j * L, L)] = jnp.where(nonempty, maxs[j], 0.0)
      return (i + 1, b_i,
              tuple(zero for _ in range(FV)), tuple(ninf for _ in range(FV)))

    i, g, sums, maxs = lax.while_loop(seg_cond, seg_fin, (i, g, sums, maxs))
    sums, maxs = accum_rows(buf, jnp.maximum(g, start0) - start, ghi - start,
                            sums, maxs)
    return (i, jnp.maximum(g, ghi), sums, maxs)

  def blk_pair(k2, carry):
    k = k2 * 2
    carry = lax.cond(k < nblk,
                     lambda c: process_block(k, 0, c), lambda c: c, carry)
    carry = lax.cond(k + 1 < nblk,
                     lambda c: process_block(k + 1, 1, c), lambda c: c, carry)
    return carry

  carry0 = (jnp.int32(0), a_all,
            tuple(zero for _ in range(FV)), tuple(ninf for _ in range(FV)))
  npair = lax.div(nblk + 1, 2)
  lax.fori_loop(0, npair, blk_pair, carry0)

  pltpu.sync_copy(stage, out_hbm.at[pl.ds(s0, SEG_PER_W)])


@jax.jit
def _pooled(x, batch):
  mesh = plsc.VectorSubcoreMesh(core_axis_name="c", subcore_axis_name="s",
                                num_cores=NC, num_subcores=NS)
  fn = pl.kernel(
      _body,
      out_type=jax.ShapeDtypeStruct((NSEG, 2 * F), jnp.float32),
      mesh=mesh,
      scratch_types=[
          pltpu.VMEM((SLICE_PAD,), jnp.int32),
          pltpu.VMEM((SLICE_PAD,), jnp.int32),
          pltpu.VMEM((2 * L,), jnp.int32),
          pltpu.VMEM((BLK, F), jnp.float32),
          pltpu.VMEM((BLK, F), jnp.float32),
          pltpu.VMEM((SEG_PER_W, 2 * F), jnp.float32),
          pltpu.SemaphoreType.DMA,
          pltpu.SemaphoreType.DMA,
      ],
      compiler_params=pltpu.CompilerParams(needs_layout_passes=False),
  )
  return fn(x, batch)


def kernel(x, batch):
  bpad = jnp.pad(batch.astype(jnp.int32), (0, BATCH_LEN - N_ROWS))
  return _pooled(x, bpad)
